# Initial kernel scaffold; baseline (speedup 1.0000x reference)
import jax
import jax.numpy as jnp
from jax.experimental import pallas as pl


def kernel(x, edge_index, W_rel1, b_rel1, W_root1, gamma1, beta1, W_rel2, b_rel2, W_root2, gamma2, beta2):
    raise NotImplementedError("write your pallas kernel here")


# speedup vs baseline: 4.1434x; 4.1434x over previous
import functools

import jax
import jax.numpy as jnp
from jax import lax
from jax.experimental import pallas as pl
from jax.experimental.pallas import tpu as pltpu
from jax.experimental.pallas import tpu_sc as plsc

N_NODES = 10000
D = 128
EPS = 1e-5

NC = 2
NS = 16
NW = NC * NS
CHUNK = 128
N_PAD = 10112
ROWS_PER_TILE = N_PAD // NS


def _make_sc_agg(e_pad: int):
    chunks_per_tile = e_pad // (NW * CHUNK)
    edges_per_tile = chunks_per_tile * CHUNK
    mesh = plsc.VectorSubcoreMesh(
        core_axis_name="c", subcore_axis_name="s", num_cores=NC, num_subcores=NS
    )

    @functools.partial(
        pl.kernel,
        out_type=jax.ShapeDtypeStruct((NC, N_PAD, D), jnp.float32),
        mesh=mesh,
        scratch_types=[
            pltpu.VMEM((CHUNK,), jnp.int32),
            pltpu.VMEM((CHUNK,), jnp.int32),
            pltpu.VMEM((CHUNK, D), jnp.float32),
            pltpu.VMEM_SHARED((N_PAD, D), jnp.float32),
            pltpu.SemaphoreType.DMA,
        ],
    )
    def sc_agg(x_hbm, src_hbm, dst_hbm, zeros_hbm, out_hbm,
               idx_s, idx_d, rows, agg_sh, sem):
        c = lax.axis_index("c")
        s = lax.axis_index("s")
        wid = c * NS + s
        r0 = s * ROWS_PER_TILE
        pltpu.sync_copy(zeros_hbm.at[pl.ds(r0, ROWS_PER_TILE)],
                        agg_sh.at[pl.ds(r0, ROWS_PER_TILE)])
        plsc.subcore_barrier()

        ebase = wid * edges_per_tile

        @pl.loop(0, chunks_per_tile)
        def _(i):
            b = ebase + i * CHUNK
            pltpu.sync_copy(src_hbm.at[pl.ds(b, CHUNK)], idx_s)
            pltpu.sync_copy(dst_hbm.at[pl.ds(b, CHUNK)], idx_d)
            pltpu.async_copy(x_hbm.at[idx_s], rows, sem).wait()
            pltpu.sync_copy(rows, agg_sh.at[idx_d], add=True)

        plsc.subcore_barrier()
        pltpu.sync_copy(agg_sh.at[pl.ds(r0, ROWS_PER_TILE)],
                        out_hbm.at[c, pl.ds(r0, ROWS_PER_TILE)])

    return sc_agg


def _make_tc_layer(relu: bool, pad_out: bool):
    def body(p_ref, x_ref, wrel_ref, brel_ref, wroot_ref, gamma_ref, beta_ref,
             o_ref):
        agg = p_ref[0, :N_NODES, :] + p_ref[1, :N_NODES, :]
        x = x_ref[:N_NODES, :]
        h = jnp.dot(agg, wrel_ref[...], preferred_element_type=jnp.float32)
        h = h + jnp.dot(x, wroot_ref[...], preferred_element_type=jnp.float32)
        h = h + brel_ref[...]
        mu = jnp.mean(h, axis=0, keepdims=True)
        var = jnp.mean(jnp.square(h - mu), axis=0, keepdims=True)
        hn = (h - mu) * lax.rsqrt(var + EPS) * gamma_ref[...] + beta_ref[...]
        if relu:
            hn = jnp.maximum(hn, 0.0)
        if pad_out:
            o_ref[:N_NODES, :] = hn
            o_ref[N_NODES:, :] = jnp.zeros((N_PAD - N_NODES, D), jnp.float32)
        else:
            o_ref[...] = hn

    out_rows = N_PAD if pad_out else N_NODES
    return pl.pallas_call(
        body,
        out_shape=jax.ShapeDtypeStruct((out_rows, D), jnp.float32),
    )


def kernel(x, edge_index, W_rel1, b_rel1, W_root1, gamma1, beta1,
           W_rel2, b_rel2, W_root2, gamma2, beta2):
    src = edge_index[0].astype(jnp.int32)
    dst = edge_index[1].astype(jnp.int32)
    e = src.shape[0]
    grain = NW * CHUNK
    e_pad = ((e + grain - 1) // grain) * grain
    src_p = jnp.full((e_pad,), N_NODES, jnp.int32).at[:e].set(src)
    dst_p = jnp.full((e_pad,), N_NODES, jnp.int32).at[:e].set(dst)

    x_pad = jnp.zeros((N_PAD, D), jnp.float32).at[:N_NODES].set(x)
    zeros = jnp.zeros((N_PAD, D), jnp.float32)

    sc_agg = _make_sc_agg(e_pad)
    tc1 = _make_tc_layer(relu=True, pad_out=True)
    tc2 = _make_tc_layer(relu=False, pad_out=False)

    b1 = b_rel1.reshape(1, D)
    g1 = gamma1.reshape(1, D)
    be1 = beta1.reshape(1, D)
    b2 = b_rel2.reshape(1, D)
    g2 = gamma2.reshape(1, D)
    be2 = beta2.reshape(1, D)

    p1 = sc_agg(x_pad, src_p, dst_p, zeros)
    h1 = tc1(p1, x_pad, W_rel1, b1, W_root1, g1, be1)
    p2 = sc_agg(h1, src_p, dst_p, zeros)
    h2 = tc2(p2, h1, W_rel2, b2, W_root2, g2, be2)
    return h2
